# baseline (speedup 1.0000x reference)
import functools

import jax
import jax.numpy as jnp
from jax import lax
from jax.experimental import pallas as pl
from jax.experimental.pallas import tpu as pltpu
from jax.experimental.pallas import tpu_sc as plsc

_NBUF = 2
_ITEM = 512
_NTC = _ITEM // 128


def _gather_kernel(n_rows, d_model, items_per_w, nc):
    mesh = plsc.VectorSubcoreMesh(core_axis_name="c", subcore_axis_name="s")
    hist = n_rows // 4096
    n_tr = d_model // 8
    qmax = 4096 // _ITEM
    n_super = items_per_w // _NBUF

    @functools.partial(
        pl.kernel,
        mesh=mesh,
        out_type=jax.ShapeDtypeStruct((hist, n_tr, 32, 8, 128), jnp.float32),
        compiler_params=pltpu.CompilerParams(
            use_tc_tiling_on_sc=False, needs_layout_passes=False
        ),
        scratch_types=[
            pltpu.VMEM((_NBUF, _ITEM), jnp.int32),
            pltpu.VMEM((_NBUF, _ITEM, d_model), jnp.float32),
            pltpu.VMEM((n_tr, _NTC, 8, 128), jnp.float32),
            [pltpu.SemaphoreType.DMA] * _NBUF,
            [pltpu.SemaphoreType.DMA] * _NBUF,
            pltpu.SemaphoreType.DMA,
        ],
    )
    def k(idx_hbm, table_hbm, out_hbm, idx_v, rows_v, slab_v, isems, gsems, ssem):
        wid = lax.axis_index("s") * nc + lax.axis_index("c")
        t0 = wid * items_per_w
        iota16 = lax.iota(jnp.int32, 16)

        for b in range(_NBUF):
            pltpu.async_copy(
                idx_hbm.at[pl.ds((t0 + b) * _ITEM, _ITEM)], idx_v.at[b], isems[b]
            )
        for b in range(_NBUF):
            pltpu.make_async_copy(
                idx_hbm.at[pl.ds((t0 + b) * _ITEM, _ITEM)], idx_v.at[b], isems[b]
            ).wait()
            pltpu.async_copy(table_hbm.at[idx_v.at[b]], rows_v.at[b], gsems[b])

        def slot_step(g, b):
            j = g * _NBUF + b
            t = t0 + j

            pltpu.make_async_copy(
                table_hbm.at[idx_v.at[b]], rows_v.at[b], gsems[b]
            ).wait()

            @pl.when(j + _NBUF < items_per_w)
            def _prefetch_idx(b=b):
                pltpu.async_copy(
                    idx_hbm.at[pl.ds((t + _NBUF) * _ITEM, _ITEM)],
                    idx_v.at[b],
                    isems[b],
                )

            @pl.when(j > 0)
            def _wait_prev_store():
                pltpu.make_async_copy(
                    slab_v, out_hbm.at[0, :, pl.ds(0, _NTC)], ssem
                ).wait()

            for tr in range(n_tr):
                for tc in range(_NTC):
                    for r in range(8):
                        d = tr * 8 + r
                        colsel = jnp.full((16,), d, jnp.int32)

                        def cc_body(cc, c2, tr=tr, tc=tc, r=r, b=b,
                                    colsel=colsel):
                            rowsel = iota16 + (tc * 128 + cc * 16)
                            vals = plsc.load_gather(
                                rows_v.at[b], [rowsel, colsel]
                            )
                            slab_v[tr, tc, r, pl.ds(cc * 16, 16)] = vals
                            return c2

                        lax.fori_loop(0, 8, cc_body, 0)

            h = t // qmax
            q = lax.rem(t, qmax)
            pltpu.async_copy(
                slab_v, out_hbm.at[h, :, pl.ds(q * _NTC, _NTC)], ssem
            )

            @pl.when(j + _NBUF < items_per_w)
            def _next_gather(b=b):
                pltpu.make_async_copy(
                    idx_hbm.at[pl.ds((t + _NBUF) * _ITEM, _ITEM)],
                    idx_v.at[b],
                    isems[b],
                ).wait()
                pltpu.async_copy(table_hbm.at[idx_v.at[b]], rows_v.at[b], gsems[b])

        def body(g, carry):
            for b in range(_NBUF):
                slot_step(g, b)
            return carry

        lax.fori_loop(0, n_super, body, 0)
        pltpu.make_async_copy(
            slab_v, out_hbm.at[0, :, pl.ds(0, _NTC)], ssem
        ).wait()

    return k


def kernel(x, table):
    batch, hist = x.shape
    vocab, d_model = table.shape
    n_rows = batch * hist

    info = plsc.get_sparse_core_info()
    nw = info.num_cores * info.num_subcores
    items_per_w = n_rows // (_ITEM * nw)

    idx = jnp.transpose(x).reshape(n_rows).astype(jnp.int32)
    k = _gather_kernel(n_rows, d_model, items_per_w, info.num_cores)
    p = k(idx, table)
    out = jnp.reshape(jnp.transpose(p, (2, 4, 0, 1, 3)), (batch, hist, d_model))
    return out
